# Initial kernel scaffold; baseline (speedup 1.0000x reference)
import jax
import jax.numpy as jnp
from jax.experimental import pallas as pl


def kernel(x, edge_attr, edge_index, eb_W1, eb_b1, eb_W2, eb_b2, eb_g, eb_beta, nb_W1, nb_b1, nb_W2, nb_b2, nb_g, nb_beta):
    raise NotImplementedError("write your pallas kernel here")


# speedup vs baseline: 3.4104x; 3.4104x over previous
import functools

import jax
import jax.numpy as jnp
from jax import lax
from jax.experimental import pallas as pl
from jax.experimental.pallas import tpu as pltpu
from jax.experimental.pallas import tpu_sc as plsc

N = 10000
E = 320000
H = 128

NC = 2
NS = 16
NW = NC * NS
L = 16

CH = 80
PER_W = E // NW
NCHUNK = PER_W // CH
ROWS_PER_TILE = N // 10

_EPS = 1e-5


def _ln(h, g, b):
    m = jnp.mean(h, axis=-1, keepdims=True)
    v = jnp.mean((h - m) * (h - m), axis=-1, keepdims=True)
    return (h - m) * lax.rsqrt(v + _EPS) * g + b


def _pre_body(x_ref, w1a_ref, w1b_ref, b1_ref, xs_ref, xr_ref):
    xb = x_ref[...]
    xs_ref[...] = jnp.dot(xb, w1a_ref[...], preferred_element_type=jnp.float32)
    xr_ref[...] = (jnp.dot(xb, w1b_ref[...], preferred_element_type=jnp.float32)
                   + b1_ref[...])


def _edge_body(g_ref, ea_ref, w1c_ref, w2_ref, b2_ref, gam_ref, bet_ref,
               enew_ref, eout_ref):
    ea = ea_ref[...]
    h = jnp.maximum(
        g_ref[...] + jnp.dot(ea, w1c_ref[...], preferred_element_type=jnp.float32),
        0.0)
    h2 = jnp.dot(h, w2_ref[...], preferred_element_type=jnp.float32) + b2_ref[...]
    en = _ln(h2, gam_ref[...], bet_ref[...])
    enew_ref[...] = en
    eout_ref[...] = ea + en


def _node_body(x_ref, p0_ref, p1_ref, w1x_ref, w1a_ref, b1_ref, w2_ref, b2_ref,
               gam_ref, bet_ref, out_ref):
    xb = x_ref[...]
    agg = p0_ref[...] + p1_ref[...]
    h = jnp.maximum(
        jnp.dot(xb, w1x_ref[...], preferred_element_type=jnp.float32)
        + jnp.dot(agg, w1a_ref[...], preferred_element_type=jnp.float32)
        + b1_ref[...],
        0.0)
    h2 = jnp.dot(h, w2_ref[...], preferred_element_type=jnp.float32) + b2_ref[...]
    out_ref[...] = xb + _ln(h2, gam_ref[...], bet_ref[...])


def _full(shape):
    return pl.BlockSpec(shape, lambda i: (0,) * len(shape))


_MESH = plsc.VectorSubcoreMesh(core_axis_name="c", subcore_axis_name="s")


@functools.partial(
    pl.kernel,
    out_type=jax.ShapeDtypeStruct((E, H), jnp.float32),
    mesh=_MESH,
    scratch_types=[
        pltpu.VMEM((NCHUNK, CH), jnp.int32),
        pltpu.VMEM((NCHUNK, CH), jnp.int32),
        pltpu.VMEM((CH, H), jnp.float32),
        pltpu.VMEM((CH, H), jnp.float32),
        pltpu.SemaphoreType.DMA,
    ],
)
def _gather_combine(xs_hbm, xr_hbm, sidx_hbm, ridx_hbm, g_hbm,
                    sidx_v, ridx_v, buf_a, buf_b, sem):
    wid = lax.axis_index("s") * NC + lax.axis_index("c")
    row0 = wid * NCHUNK
    pltpu.sync_copy(sidx_hbm.at[wid], sidx_v)
    pltpu.sync_copy(ridx_hbm.at[wid], ridx_v)

    def chunk(j, carry):
        base = (row0 + j) * CH
        ca = pltpu.async_copy(xs_hbm.at[sidx_v.at[j]], buf_a, sem)
        cb = pltpu.async_copy(xr_hbm.at[ridx_v.at[j]], buf_b, sem)
        ca.wait()
        cb.wait()

        def add_row(i, c):
            for k in range(H // L):
                sl = pl.ds(k * L, L)
                buf_a[i, sl] = buf_a[i, sl] + buf_b[i, sl]
            return c

        lax.fori_loop(0, CH, add_row, 0)
        pltpu.sync_copy(buf_a, g_hbm.at[pl.ds(base, CH)])
        return carry

    lax.fori_loop(0, NCHUNK, chunk, 0)


@functools.partial(
    pl.kernel,
    out_type=jax.ShapeDtypeStruct((NC, N, H), jnp.float32),
    mesh=_MESH,
    scratch_types=[
        pltpu.VMEM((NCHUNK, CH), jnp.int32),
        pltpu.VMEM((CH, H), jnp.float32),
        pltpu.VMEM((ROWS_PER_TILE // 25, H), jnp.float32),
        pltpu.VMEM_SHARED((N, H), jnp.float32),
        pltpu.SemaphoreType.DMA,
    ],
)
def _scatter_add(enew_hbm, ridx_hbm, out_hbm, ridx_v, rows_v, zbuf, acc, sem):
    cid = lax.axis_index("c")
    sid = lax.axis_index("s")
    wid = sid * NC + cid
    row0 = wid * NCHUNK

    zrows = ROWS_PER_TILE // 25

    def zero_row(i, c):
        for k in range(H // L):
            zbuf[i, pl.ds(k * L, L)] = jnp.zeros((L,), jnp.float32)
        return c

    lax.fori_loop(0, zrows, zero_row, 0)

    @pl.when(sid < 10)
    def _():
        for t in range(25):
            pltpu.sync_copy(
                zbuf, acc.at[pl.ds(sid * ROWS_PER_TILE + t * zrows, zrows)])

    plsc.subcore_barrier()

    pltpu.sync_copy(ridx_hbm.at[wid], ridx_v)

    def chunk(j, carry):
        base = (row0 + j) * CH
        pltpu.async_copy(enew_hbm.at[pl.ds(base, CH)], rows_v, sem).wait()
        pltpu.sync_copy(rows_v, acc.at[ridx_v.at[j]], add=True)
        return carry

    lax.fori_loop(0, NCHUNK, chunk, 0)
    plsc.subcore_barrier()

    @pl.when(sid < 10)
    def _():
        pltpu.sync_copy(acc.at[pl.ds(sid * ROWS_PER_TILE, ROWS_PER_TILE)],
                        out_hbm.at[cid, pl.ds(sid * ROWS_PER_TILE, ROWS_PER_TILE)])


def kernel(x, edge_attr, edge_index, eb_W1, eb_b1, eb_W2, eb_b2, eb_g, eb_beta,
           nb_W1, nb_b1, nb_W2, nb_b2, nb_g, nb_beta):
    senders = edge_index[0].astype(jnp.int32).reshape(NW, NCHUNK, CH)
    receivers = edge_index[1].astype(jnp.int32).reshape(NW, NCHUNK, CH)

    w1a, w1b, w1c = eb_W1[:H], eb_W1[H:2 * H], eb_W1[2 * H:]
    nw1x, nw1a = nb_W1[:H], nb_W1[H:]

    nb_blk = 1000
    xs, xr = pl.pallas_call(
        _pre_body,
        grid=(N // nb_blk,),
        in_specs=[
            pl.BlockSpec((nb_blk, H), lambda i: (i, 0)),
            _full((H, H)), _full((H, H)), _full((H,)),
        ],
        out_specs=[pl.BlockSpec((nb_blk, H), lambda i: (i, 0))] * 2,
        out_shape=[jax.ShapeDtypeStruct((N, H), jnp.float32)] * 2,
    )(x, w1a, w1b, eb_b1)

    g = _gather_combine(xs, xr, senders, receivers)

    eb_blk = 2000
    e_new, e_out = pl.pallas_call(
        _edge_body,
        grid=(E // eb_blk,),
        in_specs=[
            pl.BlockSpec((eb_blk, H), lambda i: (i, 0)),
            pl.BlockSpec((eb_blk, H), lambda i: (i, 0)),
            _full((H, H)), _full((H, H)), _full((H,)), _full((H,)), _full((H,)),
        ],
        out_specs=[pl.BlockSpec((eb_blk, H), lambda i: (i, 0))] * 2,
        out_shape=[jax.ShapeDtypeStruct((E, H), jnp.float32)] * 2,
    )(g, edge_attr, w1c, eb_W2, eb_b2, eb_g, eb_beta)

    partial = _scatter_add(e_new, receivers)

    x_out = pl.pallas_call(
        _node_body,
        grid=(N // nb_blk,),
        in_specs=[
            pl.BlockSpec((nb_blk, H), lambda i: (i, 0)),
            pl.BlockSpec((nb_blk, H), lambda i: (i, 0)),
            pl.BlockSpec((nb_blk, H), lambda i: (i, 0)),
            _full((H, H)), _full((H, H)), _full((H,)),
            _full((H, H)), _full((H,)), _full((H,)), _full((H,)),
        ],
        out_specs=pl.BlockSpec((nb_blk, H), lambda i: (i, 0)),
        out_shape=jax.ShapeDtypeStruct((N, H), jnp.float32),
    )(x, partial[0], partial[1], nw1x, nw1a, nb_b1, nb_W2, nb_b2, nb_g, nb_beta)

    return (x_out, e_out)
